# baseline (speedup 1.0000x reference)
import functools

import jax
import jax.numpy as jnp
from jax import lax
from jax.experimental import pallas as pl
from jax.experimental.pallas import tpu as pltpu
from jax.experimental.pallas import tpu_sc as plsc

NP = 10240
EP = 327680
D = 128

_CH = 128
_RPT = NP // 16
_NCH = EP // _CH // 32
_NCHA = EP // _CH // 16

_mesh = lambda: plsc.VectorSubcoreMesh(core_axis_name="c", subcore_axis_name="s")
_SC_PARAMS = pltpu.CompilerParams(use_tc_tiling_on_sc=False)


@functools.partial(
    pl.kernel,
    out_type=jax.ShapeDtypeStruct((2 * NP,), jnp.float32),
    mesh=_mesh(),
    scratch_types=[
        pltpu.VMEM((_NCH, _CH), jnp.int32),
        pltpu.VMEM((_CH,), jnp.float32),
        pltpu.VMEM((_RPT,), jnp.float32),
        pltpu.VMEM_SHARED((NP,), jnp.float32),
        pltpu.SemaphoreType.DMA,
    ],
    compiler_params=_SC_PARAMS,
)
def _deg(dst2_hbm, out_hbm, idx_d, ones_v, init_v, acc, ssem):
    c = lax.axis_index("c")
    s = lax.axis_index("s")
    wid = c * 16 + s

    def fill_ones(i, _):
        ones_v[pl.ds(i * 16, 16)] = jnp.ones((16,), jnp.float32)
        return 0

    lax.fori_loop(0, _CH // 16, fill_ones, 0)

    initval = jnp.where(c == 0, jnp.float32(1.0), jnp.float32(0.0))

    def fill_init(i, _):
        init_v[pl.ds(i * 16, 16)] = jnp.ones((16,), jnp.float32) * initval
        return 0

    lax.fori_loop(0, _RPT // 16, fill_init, 0)
    pltpu.sync_copy(dst2_hbm.at[pl.ds(wid * _NCH, _NCH)], idx_d)
    pltpu.sync_copy(init_v, acc.at[pl.ds(s * _RPT, _RPT)])
    plsc.subcore_barrier()

    def fire(g, _):
        pltpu.async_copy(ones_v, acc.at[idx_d.at[g]], ssem, add=True)
        return 0

    lax.fori_loop(0, _NCH, fire, 0)

    def drain(g, _):
        pltpu.make_async_copy(ones_v, acc.at[idx_d.at[0]], ssem).wait()
        return 0

    lax.fori_loop(0, _NCH, drain, 0)
    plsc.subcore_barrier()
    pltpu.sync_copy(acc.at[pl.ds(s * _RPT, _RPT)],
                    out_hbm.at[pl.ds(c * NP + s * _RPT, _RPT)])


def _make_agg(Hc, NB):
    @functools.partial(
        pl.kernel,
        out_type=jax.ShapeDtypeStruct((NP, D), jnp.float32),
        mesh=_mesh(),
        scratch_types=[
            pltpu.VMEM((_NCHA, _CH), jnp.int32),
            pltpu.VMEM((_NCHA, _CH), jnp.int32),
            [pltpu.VMEM((_CH, Hc), jnp.float32) for _ in range(NB)],
            pltpu.VMEM_SHARED((NP, Hc), jnp.float32),
            pltpu.VMEM_SHARED((NP, Hc), jnp.float32),
            pltpu.SemaphoreType.DMA,
            [pltpu.SemaphoreType.DMA for _ in range(NB)],
        ],
        compiler_params=_SC_PARAMS,
    )
    def agg(ts_hbm, src2_hbm, dst2_hbm, out_hbm, idx_s, idx_d, rows, ts_s,
            acc, gsem, ssems):
        c = lax.axis_index("c")
        s = lax.axis_index("s")
        r0 = s * _RPT
        c0 = c * Hc
        pds = [
            pltpu.async_copy(ts_hbm.at[pl.ds(r0, _RPT), pl.ds(c0, Hc)],
                             acc.at[pl.ds(r0, _RPT)], gsem),
            pltpu.async_copy(ts_hbm.at[pl.ds(r0, _RPT), pl.ds(c0, Hc)],
                             ts_s.at[pl.ds(r0, _RPT)], gsem),
            pltpu.async_copy(src2_hbm.at[pl.ds(s * _NCHA, _NCHA)], idx_s,
                             gsem),
            pltpu.async_copy(dst2_hbm.at[pl.ds(s * _NCHA, _NCHA)], idx_d,
                             gsem),
        ]
        for d in pds:
            d.wait()
        plsc.subcore_barrier()

        def group(g, _):
            base = g * NB
            gds = []
            for b in range(NB):
                @pl.when(g > 0)
                def _(b=b):
                    pltpu.make_async_copy(
                        rows[b], acc.at[idx_d.at[base]], ssems[b]).wait()
                gds.append(pltpu.async_copy(ts_s.at[idx_s.at[base + b]],
                                            rows[b], gsem))
            for b in range(NB):
                gds[b].wait()
                pltpu.async_copy(rows[b], acc.at[idx_d.at[base + b]],
                                 ssems[b], add=True)
            return 0

        lax.fori_loop(0, _NCHA // NB, group, 0)
        for b in range(NB):
            pltpu.make_async_copy(rows[b], acc.at[idx_d.at[0]],
                                  ssems[b]).wait()
        plsc.subcore_barrier()
        pltpu.sync_copy(acc.at[pl.ds(r0, _RPT)],
                        out_hbm.at[pl.ds(r0, _RPT), pl.ds(c0, Hc)])

    return agg


def _make_agg_stream(Hc, NB):
    NGRP = _NCHA // NB
    assert NGRP % 2 == 0
    NROWS = EP // _CH

    @functools.partial(
        pl.kernel,
        out_type=jax.ShapeDtypeStruct((NP, D), jnp.float32),
        mesh=_mesh(),
        scratch_types=[
            pltpu.VMEM((2, NB, _CH), jnp.int32),
            pltpu.VMEM((2, NB, _CH), jnp.int32),
            [pltpu.VMEM((_CH, Hc), jnp.float32) for _ in range(NB)],
            pltpu.VMEM_SHARED((NP, Hc), jnp.float32),
            pltpu.VMEM_SHARED((NP, Hc), jnp.float32),
            pltpu.SemaphoreType.DMA,
            pltpu.SemaphoreType.DMA,
            [pltpu.SemaphoreType.DMA for _ in range(NB)],
        ],
        compiler_params=_SC_PARAMS,
    )
    def agg(ts_hbm, src2_hbm, dst2_hbm, out_hbm, idx_s, idx_d, rows, ts_s,
            acc, gsem, isem, ssems):
        c = lax.axis_index("c")
        s = lax.axis_index("s")
        r0 = s * _RPT
        c0 = c * Hc
        row0 = s * _NCHA
        pds = [
            pltpu.async_copy(ts_hbm.at[pl.ds(r0, _RPT), pl.ds(c0, Hc)],
                             acc.at[pl.ds(r0, _RPT)], gsem),
            pltpu.async_copy(ts_hbm.at[pl.ds(r0, _RPT), pl.ds(c0, Hc)],
                             ts_s.at[pl.ds(r0, _RPT)], gsem),
        ]
        pltpu.async_copy(src2_hbm.at[pl.ds(row0, NB)], idx_s.at[0], isem)
        pltpu.async_copy(dst2_hbm.at[pl.ds(row0, NB)], idx_d.at[0], isem)
        for d in pds:
            d.wait()
        plsc.subcore_barrier()

        def pair(q, _):
            for par in (0, 1):
                g = 2 * q + par
                nxt = 1 - par
                for b in range(NB):
                    @pl.when(g > 0)
                    def _(b=b):
                        pltpu.make_async_copy(
                            rows[b], acc.at[idx_d.at[par, 0]],
                            ssems[b]).wait()
                pltpu.make_async_copy(src2_hbm.at[pl.ds(0, NB)],
                                      idx_s.at[par], isem).wait()
                pltpu.make_async_copy(dst2_hbm.at[pl.ds(0, NB)],
                                      idx_d.at[par], isem).wait()
                nrow = jnp.minimum(row0 + (g + 1) * NB, NROWS - NB)
                pltpu.async_copy(src2_hbm.at[pl.ds(nrow, NB)],
                                 idx_s.at[nxt], isem)
                pltpu.async_copy(dst2_hbm.at[pl.ds(nrow, NB)],
                                 idx_d.at[nxt], isem)
                gds = [pltpu.async_copy(ts_s.at[idx_s.at[par, b]], rows[b],
                                        gsem) for b in range(NB)]
                for b in range(NB):
                    gds[b].wait()
                    pltpu.async_copy(rows[b], acc.at[idx_d.at[par, b]],
                                     ssems[b], add=True)
            return 0

        lax.fori_loop(0, NGRP // 2, pair, 0)
        for b in range(NB):
            pltpu.make_async_copy(rows[b], acc.at[idx_d.at[0, 0]],
                                  ssems[b]).wait()
        pltpu.make_async_copy(src2_hbm.at[pl.ds(0, NB)], idx_s.at[0],
                              isem).wait()
        pltpu.make_async_copy(dst2_hbm.at[pl.ds(0, NB)], idx_d.at[0],
                              isem).wait()
        plsc.subcore_barrier()
        pltpu.sync_copy(acc.at[pl.ds(r0, _RPT)],
                        out_hbm.at[pl.ds(r0, _RPT), pl.ds(c0, Hc)])

    return agg


_agg128 = _make_agg_stream(64, 5)
_agg64 = _make_agg(32, 10)
_agg32 = _make_agg(16, 20)


_R = 1024
_GRID = NP // _R
_BN_C = 0.9999950000374997

_deg_specs = [
    pl.BlockSpec((_R,), lambda i: (i,)),
    pl.BlockSpec((_R,), lambda i: (NP // _R + i,)),
]


def _pad_cols(t):
    w = t.shape[1]
    if w == D:
        return t
    return jnp.concatenate(
        [t, jnp.zeros((t.shape[0], D - w), jnp.float32)], axis=1)


def _stage0(x, degp, ln_g, ln_b, W1):
    def body(x_ref, d0_ref, d1_ref, g_ref, b_ref, w_ref, o_ref):
        xb = x_ref[...]
        mu = jnp.mean(xb, axis=1, keepdims=True)
        var = jnp.mean((xb - mu) ** 2, axis=1, keepdims=True)
        h = (xb - mu) * lax.rsqrt(var + 1e-5) * g_ref[...] + b_ref[...]
        dinv = lax.rsqrt(d0_ref[...] + d1_ref[...])[:, None]
        t = jnp.dot(h, w_ref[...], preferred_element_type=jnp.float32)
        o_ref[...] = t * dinv

    return pl.pallas_call(
        body,
        grid=(_GRID,),
        in_specs=[
            pl.BlockSpec((_R, D), lambda i: (i, 0)),
            *_deg_specs,
            pl.BlockSpec((D,), lambda i: (0,)),
            pl.BlockSpec((D,), lambda i: (0,)),
            pl.BlockSpec((D, D), lambda i: (0, 0)),
        ],
        out_specs=pl.BlockSpec((_R, D), lambda i: (i, 0)),
        out_shape=jax.ShapeDtypeStruct((NP, D), jnp.float32),
    )(x, degp, degp, ln_g, ln_b, W1)


def _stage_mid(part, degp, b, g, be, Wn):
    Hin, Hout = Wn.shape[0], Wn.shape[1]

    def body(p_ref, d0_ref, d1_ref, b_ref, g_ref, be_ref, w_ref, o_ref):
        dinv = lax.rsqrt(d0_ref[...] + d1_ref[...])[:, None]
        agg = p_ref[...][:, :Hin]
        y = dinv * agg + b_ref[...]
        h = jnp.maximum(y * (_BN_C * g_ref[...]) + be_ref[...], 0.0)
        t = jnp.dot(h, w_ref[...], preferred_element_type=jnp.float32) * dinv
        o_ref[...] = _pad_cols(t)

    return pl.pallas_call(
        body,
        grid=(_GRID,),
        in_specs=[
            pl.BlockSpec((_R, D), lambda i: (i, 0)),
            *_deg_specs,
            pl.BlockSpec((Hin,), lambda i: (0,)),
            pl.BlockSpec((Hin,), lambda i: (0,)),
            pl.BlockSpec((Hin,), lambda i: (0,)),
            pl.BlockSpec((Hin, Hout), lambda i: (0, 0)),
        ],
        out_specs=pl.BlockSpec((_R, D), lambda i: (i, 0)),
        out_shape=jax.ShapeDtypeStruct((NP, D), jnp.float32),
    )(part, degp, degp, b, g, be, Wn)


def _ln_in(z, g, b):
    mu = jnp.mean(z, axis=1, keepdims=True)
    var = jnp.mean((z - mu) ** 2, axis=1, keepdims=True)
    return (z - mu) * lax.rsqrt(var + 1e-5) * g + b


def _stage4(part, degp, b4, g4, be4, Wc1, bc1, lg1, lb1,
            Wc2, bc2, lg2, lb2, Wc3, bc3):
    Hin = Wc1.shape[0]

    def body(p_ref, d0_ref, d1_ref, b_ref, g_ref, be_ref,
             w1_ref, b1_ref, g1_ref, be1_ref,
             w2_ref, b2_ref, g2_ref, be2_ref,
             w3_ref, b3_ref, o_ref):
        dinv = lax.rsqrt(d0_ref[...] + d1_ref[...])[:, None]
        agg = p_ref[...][:, :Hin]
        y = dinv * agg + b_ref[...]
        h = jnp.maximum(y * (_BN_C * g_ref[...]) + be_ref[...], 0.0)
        z = jnp.dot(h, w1_ref[...], preferred_element_type=jnp.float32) + b1_ref[...]
        h = jnp.maximum(_ln_in(z, g1_ref[...], be1_ref[...]), 0.0)
        z = jnp.dot(h, w2_ref[...], preferred_element_type=jnp.float32) + b2_ref[...]
        h = jnp.maximum(_ln_in(z, g2_ref[...], be2_ref[...]), 0.0)
        o_ref[...] = jnp.dot(h, w3_ref[...],
                             preferred_element_type=jnp.float32) + b3_ref[...]

    return pl.pallas_call(
        body,
        grid=(_GRID,),
        in_specs=[
            pl.BlockSpec((_R, D), lambda i: (i, 0)),
            *_deg_specs,
            pl.BlockSpec((Hin,), lambda i: (0,)),
            pl.BlockSpec((Hin,), lambda i: (0,)),
            pl.BlockSpec((Hin,), lambda i: (0,)),
            pl.BlockSpec((32, 16), lambda i: (0, 0)),
            pl.BlockSpec((16,), lambda i: (0,)),
            pl.BlockSpec((16,), lambda i: (0,)),
            pl.BlockSpec((16,), lambda i: (0,)),
            pl.BlockSpec((16, 8), lambda i: (0, 0)),
            pl.BlockSpec((8,), lambda i: (0,)),
            pl.BlockSpec((8,), lambda i: (0,)),
            pl.BlockSpec((8,), lambda i: (0,)),
            pl.BlockSpec((8, 8), lambda i: (0, 0)),
            pl.BlockSpec((8,), lambda i: (0,)),
        ],
        out_specs=pl.BlockSpec((_R, 8), lambda i: (i, 0)),
        out_shape=jax.ShapeDtypeStruct((NP, 8), jnp.float32),
    )(part, degp, degp, b4, g4, be4, Wc1, bc1, lg1, lb1,
      Wc2, bc2, lg2, lb2, Wc3, bc3)


def kernel(x, edge_index, ln_g, ln_b,
           W1, b1, g1, be1, W2, b2, g2, be2,
           W3, b3, g3, be3, W4, b4, g4, be4,
           Wc1, bc1, lg1, lb1, Wc2, bc2, lg2, lb2, Wc3, bc3):
    n = x.shape[0]
    x_pad = jnp.zeros((NP, D), jnp.float32).at[:n, :].set(x)
    src = edge_index[0].astype(jnp.int32)
    dst = edge_index[1].astype(jnp.int32)
    padn = EP - src.shape[0]
    pad_idx = jnp.full((padn,), NP - 1, jnp.int32)
    src_p = jnp.concatenate([src, pad_idx]).reshape(EP // _CH, _CH)
    dst_p = jnp.concatenate([dst, pad_idx]).reshape(EP // _CH, _CH)

    degp = _deg(dst_p)

    ts1 = _stage0(x_pad, degp, ln_g, ln_b, W1)
    p1 = _agg128(ts1, src_p, dst_p)
    ts2 = _stage_mid(p1, degp, b1, g1, be1, W2)
    p2 = _agg128(ts2, src_p, dst_p)
    ts3 = _stage_mid(p2, degp, b2, g2, be2, W3)
    p3 = _agg64(ts3, src_p, dst_p)
    ts4 = _stage_mid(p3, degp, b3, g3, be3, W4)
    p4 = _agg32(ts4, src_p, dst_p)
    out = _stage4(p4, degp, b4, g4, be4,
                  Wc1, bc1, lg1, lb1, Wc2, bc2, lg2, lb2, Wc3, bc3)
    return out[:n]


# speedup vs baseline: 1.0208x; 1.0208x over previous
import functools

import jax
import jax.numpy as jnp
from jax import lax
from jax.experimental import pallas as pl
from jax.experimental.pallas import tpu as pltpu
from jax.experimental.pallas import tpu_sc as plsc

NP = 10240
EP = 327680
D = 128

_CH = 128
_RPT = NP // 16
_NCH = EP // _CH // 32
_NCHA = EP // _CH // 16

_mesh = lambda: plsc.VectorSubcoreMesh(core_axis_name="c", subcore_axis_name="s")
_SC_PARAMS = pltpu.CompilerParams(use_tc_tiling_on_sc=False)


@functools.partial(
    pl.kernel,
    out_type=jax.ShapeDtypeStruct((2 * NP,), jnp.float32),
    mesh=_mesh(),
    scratch_types=[
        pltpu.VMEM((_NCH, _CH), jnp.int32),
        pltpu.VMEM((_CH,), jnp.float32),
        pltpu.VMEM((_RPT,), jnp.float32),
        pltpu.VMEM_SHARED((NP,), jnp.float32),
        pltpu.SemaphoreType.DMA,
    ],
    compiler_params=_SC_PARAMS,
)
def _deg(dst2_hbm, out_hbm, idx_d, ones_v, init_v, acc, ssem):
    c = lax.axis_index("c")
    s = lax.axis_index("s")
    wid = c * 16 + s

    def fill_ones(i, _):
        ones_v[pl.ds(i * 16, 16)] = jnp.ones((16,), jnp.float32)
        return 0

    lax.fori_loop(0, _CH // 16, fill_ones, 0)

    initval = jnp.where(c == 0, jnp.float32(1.0), jnp.float32(0.0))

    def fill_init(i, _):
        init_v[pl.ds(i * 16, 16)] = jnp.ones((16,), jnp.float32) * initval
        return 0

    lax.fori_loop(0, _RPT // 16, fill_init, 0)
    pltpu.sync_copy(dst2_hbm.at[pl.ds(wid * _NCH, _NCH)], idx_d)
    pltpu.sync_copy(init_v, acc.at[pl.ds(s * _RPT, _RPT)])
    plsc.subcore_barrier()

    def fire(g, _):
        pltpu.async_copy(ones_v, acc.at[idx_d.at[g]], ssem, add=True)
        return 0

    lax.fori_loop(0, _NCH, fire, 0)

    def drain(g, _):
        pltpu.make_async_copy(ones_v, acc.at[idx_d.at[0]], ssem).wait()
        return 0

    lax.fori_loop(0, _NCH, drain, 0)
    plsc.subcore_barrier()
    pltpu.sync_copy(acc.at[pl.ds(s * _RPT, _RPT)],
                    out_hbm.at[pl.ds(c * NP + s * _RPT, _RPT)])


def _make_agg(Hc, NB):
    @functools.partial(
        pl.kernel,
        out_type=jax.ShapeDtypeStruct((NP, D), jnp.float32),
        mesh=_mesh(),
        scratch_types=[
            pltpu.VMEM((_NCHA, _CH), jnp.int32),
            pltpu.VMEM((_NCHA, _CH), jnp.int32),
            [pltpu.VMEM((_CH, Hc), jnp.float32) for _ in range(NB)],
            pltpu.VMEM_SHARED((NP, Hc), jnp.float32),
            pltpu.VMEM_SHARED((NP, Hc), jnp.float32),
            pltpu.SemaphoreType.DMA,
            [pltpu.SemaphoreType.DMA for _ in range(NB)],
        ],
        compiler_params=_SC_PARAMS,
    )
    def agg(ts_hbm, src2_hbm, dst2_hbm, out_hbm, idx_s, idx_d, rows, ts_s,
            acc, gsem, ssems):
        c = lax.axis_index("c")
        s = lax.axis_index("s")
        r0 = s * _RPT
        c0 = c * Hc
        pds = [
            pltpu.async_copy(ts_hbm.at[pl.ds(r0, _RPT), pl.ds(c0, Hc)],
                             acc.at[pl.ds(r0, _RPT)], gsem),
            pltpu.async_copy(ts_hbm.at[pl.ds(r0, _RPT), pl.ds(c0, Hc)],
                             ts_s.at[pl.ds(r0, _RPT)], gsem),
            pltpu.async_copy(src2_hbm.at[pl.ds(s * _NCHA, _NCHA)], idx_s,
                             gsem),
            pltpu.async_copy(dst2_hbm.at[pl.ds(s * _NCHA, _NCHA)], idx_d,
                             gsem),
        ]
        for d in pds:
            d.wait()
        plsc.subcore_barrier()

        def group(g, _):
            base = g * NB
            gds = []
            for b in range(NB):
                @pl.when(g > 0)
                def _(b=b):
                    pltpu.make_async_copy(
                        rows[b], acc.at[idx_d.at[base]], ssems[b]).wait()
                gds.append(pltpu.async_copy(ts_s.at[idx_s.at[base + b]],
                                            rows[b], gsem))
            for b in range(NB):
                gds[b].wait()
                pltpu.async_copy(rows[b], acc.at[idx_d.at[base + b]],
                                 ssems[b], add=True)
            return 0

        lax.fori_loop(0, _NCHA // NB, group, 0)
        for b in range(NB):
            pltpu.make_async_copy(rows[b], acc.at[idx_d.at[0]],
                                  ssems[b]).wait()
        plsc.subcore_barrier()
        pltpu.sync_copy(acc.at[pl.ds(r0, _RPT)],
                        out_hbm.at[pl.ds(r0, _RPT), pl.ds(c0, Hc)])

    return agg


def _make_agg_stream(Hc, NB):
    NGRP = _NCHA // NB
    assert NGRP % 2 == 0
    NROWS = EP // _CH

    @functools.partial(
        pl.kernel,
        out_type=jax.ShapeDtypeStruct((NP, D), jnp.float32),
        mesh=_mesh(),
        scratch_types=[
            pltpu.VMEM((2, NB, _CH), jnp.int32),
            pltpu.VMEM((2, NB, _CH), jnp.int32),
            [pltpu.VMEM((_CH, Hc), jnp.float32) for _ in range(NB)],
            pltpu.VMEM_SHARED((NP, Hc), jnp.float32),
            pltpu.VMEM_SHARED((NP, Hc), jnp.float32),
            pltpu.SemaphoreType.DMA,
            pltpu.SemaphoreType.DMA,
            [pltpu.SemaphoreType.DMA for _ in range(NB)],
        ],
        compiler_params=_SC_PARAMS,
    )
    def agg(ts_hbm, src2_hbm, dst2_hbm, out_hbm, idx_s, idx_d, rows, ts_s,
            acc, gsem, isem, ssems):
        c = lax.axis_index("c")
        s = lax.axis_index("s")
        r0 = s * _RPT
        c0 = c * Hc
        row0 = s * _NCHA
        pds = [
            pltpu.async_copy(ts_hbm.at[pl.ds(r0, _RPT), pl.ds(c0, Hc)],
                             acc.at[pl.ds(r0, _RPT)], gsem),
            pltpu.async_copy(ts_hbm.at[pl.ds(r0, _RPT), pl.ds(c0, Hc)],
                             ts_s.at[pl.ds(r0, _RPT)], gsem),
        ]
        pltpu.async_copy(src2_hbm.at[pl.ds(row0, NB)], idx_s.at[0], isem)
        pltpu.async_copy(dst2_hbm.at[pl.ds(row0, NB)], idx_d.at[0], isem)
        for d in pds:
            d.wait()
        plsc.subcore_barrier()

        def pair(q, _):
            for par in (0, 1):
                g = 2 * q + par
                nxt = 1 - par
                for b in range(NB):
                    @pl.when(g > 0)
                    def _(b=b):
                        pltpu.make_async_copy(
                            rows[b], acc.at[idx_d.at[par, 0]],
                            ssems[b]).wait()
                pltpu.make_async_copy(src2_hbm.at[pl.ds(0, NB)],
                                      idx_s.at[par], isem).wait()
                pltpu.make_async_copy(dst2_hbm.at[pl.ds(0, NB)],
                                      idx_d.at[par], isem).wait()
                nrow = jnp.minimum(row0 + (g + 1) * NB, NROWS - NB)
                pltpu.async_copy(src2_hbm.at[pl.ds(nrow, NB)],
                                 idx_s.at[nxt], isem)
                pltpu.async_copy(dst2_hbm.at[pl.ds(nrow, NB)],
                                 idx_d.at[nxt], isem)
                gds = [pltpu.async_copy(ts_s.at[idx_s.at[par, b]], rows[b],
                                        gsem) for b in range(NB)]
                for b in range(NB):
                    gds[b].wait()
                    pltpu.async_copy(rows[b], acc.at[idx_d.at[par, b]],
                                     ssems[b], add=True)
            return 0

        lax.fori_loop(0, NGRP // 2, pair, 0)
        for b in range(NB):
            pltpu.make_async_copy(rows[b], acc.at[idx_d.at[0, 0]],
                                  ssems[b]).wait()
        pltpu.make_async_copy(src2_hbm.at[pl.ds(0, NB)], idx_s.at[0],
                              isem).wait()
        pltpu.make_async_copy(dst2_hbm.at[pl.ds(0, NB)], idx_d.at[0],
                              isem).wait()
        plsc.subcore_barrier()
        pltpu.sync_copy(acc.at[pl.ds(r0, _RPT)],
                        out_hbm.at[pl.ds(r0, _RPT), pl.ds(c0, Hc)])

    return agg


_agg128 = _make_agg_stream(64, 5)
_agg64 = _make_agg(32, 10)
_agg32 = _make_agg(16, 20)


_R = 2048
_GRID = NP // _R
_BN_C = 0.9999950000374997

_deg_specs = [
    pl.BlockSpec((_R,), lambda i: (i,)),
    pl.BlockSpec((_R,), lambda i: (NP // _R + i,)),
]


def _pad_cols(t):
    w = t.shape[1]
    if w == D:
        return t
    return jnp.concatenate(
        [t, jnp.zeros((t.shape[0], D - w), jnp.float32)], axis=1)


def _stage0(x, degp, ln_g, ln_b, W1):
    def body(x_ref, d0_ref, d1_ref, g_ref, b_ref, w_ref, o_ref):
        xb = x_ref[...]
        mu = jnp.mean(xb, axis=1, keepdims=True)
        var = jnp.mean((xb - mu) ** 2, axis=1, keepdims=True)
        h = (xb - mu) * lax.rsqrt(var + 1e-5) * g_ref[...] + b_ref[...]
        dinv = lax.rsqrt(d0_ref[...] + d1_ref[...])[:, None]
        t = jnp.dot(h, w_ref[...], preferred_element_type=jnp.float32)
        o_ref[...] = t * dinv

    return pl.pallas_call(
        body,
        grid=(_GRID,),
        in_specs=[
            pl.BlockSpec((_R, D), lambda i: (i, 0)),
            *_deg_specs,
            pl.BlockSpec((D,), lambda i: (0,)),
            pl.BlockSpec((D,), lambda i: (0,)),
            pl.BlockSpec((D, D), lambda i: (0, 0)),
        ],
        out_specs=pl.BlockSpec((_R, D), lambda i: (i, 0)),
        out_shape=jax.ShapeDtypeStruct((NP, D), jnp.float32),
    )(x, degp, degp, ln_g, ln_b, W1)


def _stage_mid(part, degp, b, g, be, Wn):
    Hin, Hout = Wn.shape[0], Wn.shape[1]

    def body(p_ref, d0_ref, d1_ref, b_ref, g_ref, be_ref, w_ref, o_ref):
        dinv = lax.rsqrt(d0_ref[...] + d1_ref[...])[:, None]
        agg = p_ref[...][:, :Hin]
        y = dinv * agg + b_ref[...]
        h = jnp.maximum(y * (_BN_C * g_ref[...]) + be_ref[...], 0.0)
        t = jnp.dot(h, w_ref[...], preferred_element_type=jnp.float32) * dinv
        o_ref[...] = _pad_cols(t)

    return pl.pallas_call(
        body,
        grid=(_GRID,),
        in_specs=[
            pl.BlockSpec((_R, D), lambda i: (i, 0)),
            *_deg_specs,
            pl.BlockSpec((Hin,), lambda i: (0,)),
            pl.BlockSpec((Hin,), lambda i: (0,)),
            pl.BlockSpec((Hin,), lambda i: (0,)),
            pl.BlockSpec((Hin, Hout), lambda i: (0, 0)),
        ],
        out_specs=pl.BlockSpec((_R, D), lambda i: (i, 0)),
        out_shape=jax.ShapeDtypeStruct((NP, D), jnp.float32),
    )(part, degp, degp, b, g, be, Wn)


def _ln_in(z, g, b):
    mu = jnp.mean(z, axis=1, keepdims=True)
    var = jnp.mean((z - mu) ** 2, axis=1, keepdims=True)
    return (z - mu) * lax.rsqrt(var + 1e-5) * g + b


def _stage4(part, degp, b4, g4, be4, Wc1, bc1, lg1, lb1,
            Wc2, bc2, lg2, lb2, Wc3, bc3):
    Hin = Wc1.shape[0]

    def body(p_ref, d0_ref, d1_ref, b_ref, g_ref, be_ref,
             w1_ref, b1_ref, g1_ref, be1_ref,
             w2_ref, b2_ref, g2_ref, be2_ref,
             w3_ref, b3_ref, o_ref):
        dinv = lax.rsqrt(d0_ref[...] + d1_ref[...])[:, None]
        agg = p_ref[...][:, :Hin]
        y = dinv * agg + b_ref[...]
        h = jnp.maximum(y * (_BN_C * g_ref[...]) + be_ref[...], 0.0)
        z = jnp.dot(h, w1_ref[...], preferred_element_type=jnp.float32) + b1_ref[...]
        h = jnp.maximum(_ln_in(z, g1_ref[...], be1_ref[...]), 0.0)
        z = jnp.dot(h, w2_ref[...], preferred_element_type=jnp.float32) + b2_ref[...]
        h = jnp.maximum(_ln_in(z, g2_ref[...], be2_ref[...]), 0.0)
        o_ref[...] = jnp.dot(h, w3_ref[...],
                             preferred_element_type=jnp.float32) + b3_ref[...]

    return pl.pallas_call(
        body,
        grid=(_GRID,),
        in_specs=[
            pl.BlockSpec((_R, D), lambda i: (i, 0)),
            *_deg_specs,
            pl.BlockSpec((Hin,), lambda i: (0,)),
            pl.BlockSpec((Hin,), lambda i: (0,)),
            pl.BlockSpec((Hin,), lambda i: (0,)),
            pl.BlockSpec((32, 16), lambda i: (0, 0)),
            pl.BlockSpec((16,), lambda i: (0,)),
            pl.BlockSpec((16,), lambda i: (0,)),
            pl.BlockSpec((16,), lambda i: (0,)),
            pl.BlockSpec((16, 8), lambda i: (0, 0)),
            pl.BlockSpec((8,), lambda i: (0,)),
            pl.BlockSpec((8,), lambda i: (0,)),
            pl.BlockSpec((8,), lambda i: (0,)),
            pl.BlockSpec((8, 8), lambda i: (0, 0)),
            pl.BlockSpec((8,), lambda i: (0,)),
        ],
        out_specs=pl.BlockSpec((_R, 8), lambda i: (i, 0)),
        out_shape=jax.ShapeDtypeStruct((NP, 8), jnp.float32),
    )(part, degp, degp, b4, g4, be4, Wc1, bc1, lg1, lb1,
      Wc2, bc2, lg2, lb2, Wc3, bc3)


def kernel(x, edge_index, ln_g, ln_b,
           W1, b1, g1, be1, W2, b2, g2, be2,
           W3, b3, g3, be3, W4, b4, g4, be4,
           Wc1, bc1, lg1, lb1, Wc2, bc2, lg2, lb2, Wc3, bc3):
    n = x.shape[0]
    x_pad = jnp.zeros((NP, D), jnp.float32).at[:n, :].set(x)
    src = edge_index[0].astype(jnp.int32)
    dst = edge_index[1].astype(jnp.int32)
    padn = EP - src.shape[0]
    pad_idx = jnp.full((padn,), NP - 1, jnp.int32)
    src_p = jnp.concatenate([src, pad_idx]).reshape(EP // _CH, _CH)
    dst_p = jnp.concatenate([dst, pad_idx]).reshape(EP // _CH, _CH)

    degp = _deg(dst_p)

    ts1 = _stage0(x_pad, degp, ln_g, ln_b, W1)
    p1 = _agg128(ts1, src_p, dst_p)
    ts2 = _stage_mid(p1, degp, b1, g1, be1, W2)
    p2 = _agg128(ts2, src_p, dst_p)
    ts3 = _stage_mid(p2, degp, b2, g2, be2, W3)
    p3 = _agg64(ts3, src_p, dst_p)
    ts4 = _stage_mid(p3, degp, b3, g3, be3, W4)
    p4 = _agg32(ts4, src_p, dst_p)
    out = _stage4(p4, degp, b4, g4, be4,
                  Wc1, bc1, lg1, lb1, Wc2, bc2, lg2, lb2, Wc3, bc3)
    return out[:n]


# speedup vs baseline: 1.0383x; 1.0171x over previous
import functools

import jax
import jax.numpy as jnp
from jax import lax
from jax.experimental import pallas as pl
from jax.experimental.pallas import tpu as pltpu
from jax.experimental.pallas import tpu_sc as plsc

NP = 10240
EP = 327680
D = 128

_CH = 128
_RPT = NP // 16
_NCH = EP // _CH // 32
_NCHA = EP // _CH // 16

_mesh = lambda: plsc.VectorSubcoreMesh(core_axis_name="c", subcore_axis_name="s")
_SC_PARAMS = pltpu.CompilerParams(use_tc_tiling_on_sc=False)


@functools.partial(
    pl.kernel,
    out_type=jax.ShapeDtypeStruct((2 * NP,), jnp.float32),
    mesh=_mesh(),
    scratch_types=[
        pltpu.VMEM((_NCH, _CH), jnp.int32),
        pltpu.VMEM((_CH,), jnp.float32),
        pltpu.VMEM((_RPT,), jnp.float32),
        pltpu.VMEM_SHARED((NP,), jnp.float32),
        pltpu.SemaphoreType.DMA,
    ],
    compiler_params=_SC_PARAMS,
)
def _deg(dst2_hbm, out_hbm, idx_d, ones_v, init_v, acc, ssem):
    c = lax.axis_index("c")
    s = lax.axis_index("s")
    wid = c * 16 + s

    def fill_ones(i, _):
        ones_v[pl.ds(i * 16, 16)] = jnp.ones((16,), jnp.float32)
        return 0

    lax.fori_loop(0, _CH // 16, fill_ones, 0)

    initval = jnp.where(c == 0, jnp.float32(1.0), jnp.float32(0.0))

    def fill_init(i, _):
        init_v[pl.ds(i * 16, 16)] = jnp.ones((16,), jnp.float32) * initval
        return 0

    lax.fori_loop(0, _RPT // 16, fill_init, 0)
    pltpu.sync_copy(dst2_hbm.at[pl.ds(wid * _NCH, _NCH)], idx_d)
    pltpu.sync_copy(init_v, acc.at[pl.ds(s * _RPT, _RPT)])
    plsc.subcore_barrier()

    def fire(g, _):
        pltpu.async_copy(ones_v, acc.at[idx_d.at[g]], ssem, add=True)
        return 0

    lax.fori_loop(0, _NCH, fire, 0)

    def drain(g, _):
        pltpu.make_async_copy(ones_v, acc.at[idx_d.at[0]], ssem).wait()
        return 0

    lax.fori_loop(0, _NCH, drain, 0)
    plsc.subcore_barrier()
    pltpu.sync_copy(acc.at[pl.ds(s * _RPT, _RPT)],
                    out_hbm.at[pl.ds(c * NP + s * _RPT, _RPT)])


def _make_agg(Hc, NB):
    @functools.partial(
        pl.kernel,
        out_type=jax.ShapeDtypeStruct((NP, D), jnp.float32),
        mesh=_mesh(),
        scratch_types=[
            pltpu.VMEM((_NCHA, _CH), jnp.int32),
            pltpu.VMEM((_NCHA, _CH), jnp.int32),
            [pltpu.VMEM((_CH, Hc), jnp.float32) for _ in range(NB)],
            pltpu.VMEM_SHARED((NP, Hc), jnp.float32),
            pltpu.VMEM_SHARED((NP, Hc), jnp.float32),
            pltpu.SemaphoreType.DMA,
            [pltpu.SemaphoreType.DMA for _ in range(NB)],
        ],
        compiler_params=_SC_PARAMS,
    )
    def agg(ts_hbm, src2_hbm, dst2_hbm, out_hbm, idx_s, idx_d, rows, ts_s,
            acc, gsem, ssems):
        c = lax.axis_index("c")
        s = lax.axis_index("s")
        r0 = s * _RPT
        c0 = c * Hc
        pds = [
            pltpu.async_copy(ts_hbm.at[pl.ds(r0, _RPT), pl.ds(c0, Hc)],
                             acc.at[pl.ds(r0, _RPT)], gsem),
            pltpu.async_copy(ts_hbm.at[pl.ds(r0, _RPT), pl.ds(c0, Hc)],
                             ts_s.at[pl.ds(r0, _RPT)], gsem),
            pltpu.async_copy(src2_hbm.at[pl.ds(s * _NCHA, _NCHA)], idx_s,
                             gsem),
            pltpu.async_copy(dst2_hbm.at[pl.ds(s * _NCHA, _NCHA)], idx_d,
                             gsem),
        ]
        for d in pds:
            d.wait()
        plsc.subcore_barrier()

        def group(g, _):
            base = g * NB
            gds = []
            for b in range(NB):
                @pl.when(g > 0)
                def _(b=b):
                    pltpu.make_async_copy(
                        rows[b], acc.at[idx_d.at[base]], ssems[b]).wait()
                gds.append(pltpu.async_copy(ts_s.at[idx_s.at[base + b]],
                                            rows[b], gsem))
            for b in range(NB):
                gds[b].wait()
                pltpu.async_copy(rows[b], acc.at[idx_d.at[base + b]],
                                 ssems[b], add=True)
            return 0

        lax.fori_loop(0, _NCHA // NB, group, 0)
        for b in range(NB):
            pltpu.make_async_copy(rows[b], acc.at[idx_d.at[0]],
                                  ssems[b]).wait()
        plsc.subcore_barrier()
        pltpu.sync_copy(acc.at[pl.ds(r0, _RPT)],
                        out_hbm.at[pl.ds(r0, _RPT), pl.ds(c0, Hc)])

    return agg


def _make_agg_stream(Hc, NB):
    NGRP = _NCHA // NB
    assert NGRP % 2 == 0
    NROWS = EP // _CH

    @functools.partial(
        pl.kernel,
        out_type=jax.ShapeDtypeStruct((NP, D), jnp.float32),
        mesh=_mesh(),
        scratch_types=[
            pltpu.VMEM((2, NB, _CH), jnp.int32),
            pltpu.VMEM((2, NB, _CH), jnp.int32),
            [pltpu.VMEM((_CH, Hc), jnp.float32) for _ in range(NB)],
            pltpu.VMEM_SHARED((NP, Hc), jnp.float32),
            pltpu.VMEM_SHARED((NP, Hc), jnp.float32),
            pltpu.SemaphoreType.DMA,
            pltpu.SemaphoreType.DMA,
            [pltpu.SemaphoreType.DMA for _ in range(NB)],
        ],
        compiler_params=_SC_PARAMS,
    )
    def agg(ts_hbm, src2_hbm, dst2_hbm, out_hbm, idx_s, idx_d, rows, ts_s,
            acc, gsem, isem, ssems):
        c = lax.axis_index("c")
        s = lax.axis_index("s")
        r0 = s * _RPT
        c0 = c * Hc
        row0 = s * _NCHA
        pds = [
            pltpu.async_copy(ts_hbm.at[pl.ds(r0, _RPT), pl.ds(c0, Hc)],
                             acc.at[pl.ds(r0, _RPT)], gsem),
            pltpu.async_copy(ts_hbm.at[pl.ds(r0, _RPT), pl.ds(c0, Hc)],
                             ts_s.at[pl.ds(r0, _RPT)], gsem),
        ]
        pltpu.async_copy(src2_hbm.at[pl.ds(row0, NB)], idx_s.at[0], isem)
        pltpu.async_copy(dst2_hbm.at[pl.ds(row0, NB)], idx_d.at[0], isem)
        for d in pds:
            d.wait()
        plsc.subcore_barrier()

        def pair(q, _):
            for par in (0, 1):
                g = 2 * q + par
                nxt = 1 - par
                for b in range(NB):
                    @pl.when(g > 0)
                    def _(b=b):
                        pltpu.make_async_copy(
                            rows[b], acc.at[idx_d.at[par, 0]],
                            ssems[b]).wait()
                pltpu.make_async_copy(src2_hbm.at[pl.ds(0, NB)],
                                      idx_s.at[par], isem).wait()
                pltpu.make_async_copy(dst2_hbm.at[pl.ds(0, NB)],
                                      idx_d.at[par], isem).wait()
                nrow = jnp.minimum(row0 + (g + 1) * NB, NROWS - NB)
                pltpu.async_copy(src2_hbm.at[pl.ds(nrow, NB)],
                                 idx_s.at[nxt], isem)
                pltpu.async_copy(dst2_hbm.at[pl.ds(nrow, NB)],
                                 idx_d.at[nxt], isem)
                gds = [pltpu.async_copy(ts_s.at[idx_s.at[par, b]], rows[b],
                                        gsem) for b in range(NB)]
                for b in range(NB):
                    gds[b].wait()
                    pltpu.async_copy(rows[b], acc.at[idx_d.at[par, b]],
                                     ssems[b], add=True)
            return 0

        lax.fori_loop(0, NGRP // 2, pair, 0)
        for b in range(NB):
            pltpu.make_async_copy(rows[b], acc.at[idx_d.at[0, 0]],
                                  ssems[b]).wait()
        pltpu.make_async_copy(src2_hbm.at[pl.ds(0, NB)], idx_s.at[0],
                              isem).wait()
        pltpu.make_async_copy(dst2_hbm.at[pl.ds(0, NB)], idx_d.at[0],
                              isem).wait()
        plsc.subcore_barrier()
        pltpu.sync_copy(acc.at[pl.ds(r0, _RPT)],
                        out_hbm.at[pl.ds(r0, _RPT), pl.ds(c0, Hc)])

    return agg


_agg128 = _make_agg_stream(64, 5)
_agg64 = _make_agg(32, 10)
_agg32 = _make_agg(16, 20)


_R = 2048
_GRID = NP // _R
_BN_C = 0.9999950000374997

_deg_specs = [
    pl.BlockSpec((_R,), lambda i: (i,)),
    pl.BlockSpec((_R,), lambda i: (NP // _R + i,)),
]


def _pad_cols(t):
    w = t.shape[1]
    if w == D:
        return t
    return jnp.concatenate(
        [t, jnp.zeros((t.shape[0], D - w), jnp.float32)], axis=1)


def _stage0(x, degp, ln_g, ln_b, W1):
    def body(x_ref, d0_ref, d1_ref, g_ref, b_ref, w_ref, o_ref):
        xb = x_ref[...]
        mu = jnp.mean(xb, axis=1, keepdims=True)
        var = jnp.mean((xb - mu) ** 2, axis=1, keepdims=True)
        h = (xb - mu) * lax.rsqrt(var + 1e-5) * g_ref[...] + b_ref[...]
        dinv = lax.rsqrt(d0_ref[...] + d1_ref[...])[:, None]
        t = jnp.dot(h, w_ref[...], preferred_element_type=jnp.float32)
        o_ref[...] = t * dinv

    return pl.pallas_call(
        body,
        grid=(_GRID,),
        in_specs=[
            pl.BlockSpec((_R, D), lambda i: (i, 0)),
            *_deg_specs,
            pl.BlockSpec((D,), lambda i: (0,)),
            pl.BlockSpec((D,), lambda i: (0,)),
            pl.BlockSpec((D, D), lambda i: (0, 0)),
        ],
        out_specs=pl.BlockSpec((_R, D), lambda i: (i, 0)),
        out_shape=jax.ShapeDtypeStruct((NP, D), jnp.float32),
    )(x, degp, degp, ln_g, ln_b, W1)


def _stage_mid(part, degp, b, g, be, Wn):
    Hin, Hout = Wn.shape[0], Wn.shape[1]

    def body(p_ref, d0_ref, d1_ref, b_ref, g_ref, be_ref, w_ref, o_ref):
        dinv = lax.rsqrt(d0_ref[...] + d1_ref[...])[:, None]
        agg = p_ref[...][:, :Hin]
        y = dinv * agg + b_ref[...]
        h = jnp.maximum(y * (_BN_C * g_ref[...]) + be_ref[...], 0.0)
        t = jnp.dot(h, w_ref[...], preferred_element_type=jnp.float32) * dinv
        o_ref[...] = _pad_cols(t)

    return pl.pallas_call(
        body,
        grid=(_GRID,),
        in_specs=[
            pl.BlockSpec((_R, D), lambda i: (i, 0)),
            *_deg_specs,
            pl.BlockSpec((Hin,), lambda i: (0,)),
            pl.BlockSpec((Hin,), lambda i: (0,)),
            pl.BlockSpec((Hin,), lambda i: (0,)),
            pl.BlockSpec((Hin, Hout), lambda i: (0, 0)),
        ],
        out_specs=pl.BlockSpec((_R, D), lambda i: (i, 0)),
        out_shape=jax.ShapeDtypeStruct((NP, D), jnp.float32),
    )(part, degp, degp, b, g, be, Wn)


def _ln_in(z, g, b):
    mu = jnp.mean(z, axis=1, keepdims=True)
    var = jnp.mean((z - mu) ** 2, axis=1, keepdims=True)
    return (z - mu) * lax.rsqrt(var + 1e-5) * g + b


def _stage4(part, degp, b4, g4, be4, Wc1, bc1, lg1, lb1,
            Wc2, bc2, lg2, lb2, Wc3, bc3):
    Hin = Wc1.shape[0]

    def body(p_ref, d0_ref, d1_ref, b_ref, g_ref, be_ref,
             w1_ref, b1_ref, g1_ref, be1_ref,
             w2_ref, b2_ref, g2_ref, be2_ref,
             w3_ref, b3_ref, o_ref):
        dinv = lax.rsqrt(d0_ref[...] + d1_ref[...])[:, None]
        agg = p_ref[...][:, :Hin]
        y = dinv * agg + b_ref[...]
        h = jnp.maximum(y * (_BN_C * g_ref[...]) + be_ref[...], 0.0)
        z = jnp.dot(h, w1_ref[...], preferred_element_type=jnp.float32) + b1_ref[...]
        h = jnp.maximum(_ln_in(z, g1_ref[...], be1_ref[...]), 0.0)
        z = jnp.dot(h, w2_ref[...], preferred_element_type=jnp.float32) + b2_ref[...]
        h = jnp.maximum(_ln_in(z, g2_ref[...], be2_ref[...]), 0.0)
        o_ref[...] = jnp.dot(h, w3_ref[...],
                             preferred_element_type=jnp.float32) + b3_ref[...]

    return pl.pallas_call(
        body,
        grid=(_GRID,),
        in_specs=[
            pl.BlockSpec((_R, D), lambda i: (i, 0)),
            *_deg_specs,
            pl.BlockSpec((Hin,), lambda i: (0,)),
            pl.BlockSpec((Hin,), lambda i: (0,)),
            pl.BlockSpec((Hin,), lambda i: (0,)),
            pl.BlockSpec((32, 16), lambda i: (0, 0)),
            pl.BlockSpec((16,), lambda i: (0,)),
            pl.BlockSpec((16,), lambda i: (0,)),
            pl.BlockSpec((16,), lambda i: (0,)),
            pl.BlockSpec((16, 8), lambda i: (0, 0)),
            pl.BlockSpec((8,), lambda i: (0,)),
            pl.BlockSpec((8,), lambda i: (0,)),
            pl.BlockSpec((8,), lambda i: (0,)),
            pl.BlockSpec((8, 8), lambda i: (0, 0)),
            pl.BlockSpec((8,), lambda i: (0,)),
        ],
        out_specs=pl.BlockSpec((_R, 8), lambda i: (i, 0)),
        out_shape=jax.ShapeDtypeStruct((NP, 8), jnp.float32),
    )(part, degp, degp, b4, g4, be4, Wc1, bc1, lg1, lb1,
      Wc2, bc2, lg2, lb2, Wc3, bc3)


def kernel(x, edge_index, ln_g, ln_b,
           W1, b1, g1, be1, W2, b2, g2, be2,
           W3, b3, g3, be3, W4, b4, g4, be4,
           Wc1, bc1, lg1, lb1, Wc2, bc2, lg2, lb2, Wc3, bc3):
    n = x.shape[0]
    x_pad = jnp.pad(x, ((0, NP - n), (0, 0)))
    padn = EP - edge_index.shape[1]
    ep = jnp.pad(edge_index.astype(jnp.int32), ((0, 0), (0, padn)),
                 constant_values=NP - 1)
    src_p = ep[0].reshape(EP // _CH, _CH)
    dst_p = ep[1].reshape(EP // _CH, _CH)

    degp = _deg(dst_p)

    ts1 = _stage0(x_pad, degp, ln_g, ln_b, W1)
    p1 = _agg128(ts1, src_p, dst_p)
    ts2 = _stage_mid(p1, degp, b1, g1, be1, W2)
    p2 = _agg128(ts2, src_p, dst_p)
    ts3 = _stage_mid(p2, degp, b2, g2, be2, W3)
    p3 = _agg64(ts3, src_p, dst_p)
    ts4 = _stage_mid(p3, degp, b3, g3, be3, W4)
    p4 = _agg32(ts4, src_p, dst_p)
    out = _stage4(p4, degp, b4, g4, be4,
                  Wc1, bc1, lg1, lb1, Wc2, bc2, lg2, lb2, Wc3, bc3)
    return out[:n]
